# baseline (speedup 1.0000x reference)
import jax
import jax.numpy as jnp
from jax import lax
from jax.experimental import pallas as pl
from jax.experimental.pallas import tpu as pltpu
from jax.experimental.pallas import tpu_sc as plsc

N = 50000
E = 800000
D = 64
H = D // 2
NSUB = 16
NP = 50176
RPT = N // NSUB
RPT_P = NP // NSUB
EPT = E // NSUB
DEG_CHUNK = 2000
PROP_CHUNK = 128
EP_CHUNKS = 6400
EPAD = EP_CHUNKS * PROP_CHUNK
CPT = EP_CHUNKS // NSUB
R = 4
GPT = CPT // R
ACC_ROWS = N + 8
CCH = 128
CTAIL = RPT - CCH * (RPT // CCH)
BLK = 1000
GRID = N // BLK


def _rsqrt_nr(d):
    i = plsc.bitcast(d, jnp.int32)
    i = jnp.int32(0x5F3759DF) - (i >> 1)
    y = plsc.bitcast(i, jnp.float32)
    for _ in range(4):
        y = y * (1.5 - 0.5 * d * y * y)
    return y


def _deg_body(src_ref, dis_ref, hist, ebuf, tmp, degl, bbuf, hists_sh):
    c = lax.axis_index("c")
    s = lax.axis_index("s")
    zeros16 = jnp.zeros((16,), jnp.float32)
    ones16 = jnp.ones((16,), jnp.float32)

    def zero_hist(i, _):
        hist[pl.ds(i * 16, 16)] = zeros16
        return 0

    lax.fori_loop(0, NP // 16, zero_hist, 0)

    def chunk(ci, _):
        pltpu.sync_copy(src_ref.at[pl.ds(s * EPT + ci * DEG_CHUNK, DEG_CHUNK)], ebuf)

        def inner(j, _):
            idx = ebuf[pl.ds(j * 16, 16)]
            plsc.addupdate_scatter(hist, [idx], ones16)
            return 0

        lax.fori_loop(0, DEG_CHUNK // 16, inner, 0)
        return 0

    lax.fori_loop(0, EPT // DEG_CHUNK, chunk, 0)

    pltpu.sync_copy(hist, hists_sh.at[pl.ds(s * NP, NP)])
    plsc.subcore_barrier()

    r0 = s * RPT_P

    def zero_deg(j, _):
        degl[pl.ds(j * 16, 16)] = zeros16
        return 0

    lax.fori_loop(0, RPT_P // 16, zero_deg, 0)

    def addk(k, _):
        pltpu.sync_copy(hists_sh.at[pl.ds(k * NP + r0, RPT_P)], tmp)

        def aj(j, _):
            sl = pl.ds(j * 16, 16)
            degl[sl] = degl[sl] + tmp[sl]
            return 0

        lax.fori_loop(0, RPT_P // 16, aj, 0)
        return 0

    lax.fori_loop(0, NSUB, addk, 0)

    def dj(j, _):
        sl = pl.ds(j * 16, 16)
        d = degl[sl]
        degl[sl] = jnp.where(d > 0.5, _rsqrt_nr(d), zeros16)
        return 0

    lax.fori_loop(0, RPT_P // 16, dj, 0)

    @pl.when(c == 0)
    def _():
        def wchunk(jc, _):
            def rowf(i, _):
                idxv = jnp.full((16,), jc * 112 + i, jnp.int32)
                dvec = plsc.load_gather(degl, [idxv])
                bbuf[i, pl.ds(0, 16)] = dvec
                bbuf[i, pl.ds(16, 16)] = dvec
                return 0

            lax.fori_loop(0, 112, rowf, 0)
            pltpu.sync_copy(bbuf, dis_ref.at[pl.ds(r0 + jc * 112, 112)])
            return 0

        lax.fori_loop(0, RPT_P // 112, wchunk, 0)


_sc_mesh = plsc.VectorSubcoreMesh(core_axis_name="c", subcore_axis_name="s")
_sc_params = pltpu.CompilerParams(needs_layout_passes=False,
                                  use_tc_tiling_on_sc=False)

_deg_call = pl.kernel(
    _deg_body,
    out_type=jax.ShapeDtypeStruct((NP, H), jnp.float32),
    mesh=_sc_mesh,
    compiler_params=_sc_params,
    scratch_types=[
        pltpu.VMEM((NP,), jnp.float32),
        pltpu.VMEM((DEG_CHUNK,), jnp.int32),
        pltpu.VMEM((RPT_P,), jnp.float32),
        pltpu.VMEM((RPT_P,), jnp.float32),
        pltpu.VMEM((112, H), jnp.float32),
        pltpu.VMEM_SHARED((NSUB * NP,), jnp.float32),
    ],
)


def _mega_body(xlo_ref, xhi_ref, dis32_ref, src2_ref, dst2_ref,
               zero_ref, slo_ref, shi_ref, ulo_ref, uhi_ref,
               acc, sidx0, didx0, sidx1, didx1,
               bb0, bb1, bb2, bb3, fb0, fb1, fb2, fb3,
               gs0, gs1, gs2, gs3, ss0, ss1, ss2, ss3, isem):
    c = lax.axis_index("c")
    s_ax = lax.axis_index("s")
    bbufs = (bb0, bb1, bb2, bb3)
    fbufs = (fb0, fb1, fb2, fb3)
    gsems = (gs0, gs1, gs2, gs3)
    ssems = (ss0, ss1, ss2, ss3)
    ILV = plsc.PackFormat.INTERLEAVED
    r0 = s_ax * RPT
    myrows = pl.ds(r0, RPT)
    c0 = s_ax * CPT
    dummy_rows = zero_ref.at[pl.ds(0, PROP_CHUNK)]
    dummy_idx = src2_ref.at[pl.ds(0, R)]

    def run(x_half, sarr, uarr):
        def chunk_body(nr, row_off, coef, dpow, s_loader, um2_base, out_base):
            sl_nr = pl.ds(0, nr)
            s_loader(row_off, nr)
            pltpu.sync_copy(dis32_ref.at[pl.ds(row_off, nr)], fb1.at[sl_nr])
            if um2_base is not None:
                pltpu.sync_copy(uarr.at[pl.ds(um2_base + row_off, nr)],
                                bb0.at[sl_nr])

            def rowfn(i, _):
                dl = fb1[i, pl.ds(0, 16)]
                dh = fb1[i, pl.ds(16, 16)]
                wl = dl * dl if dpow == 2 else dl
                wh = dh * dh if dpow == 2 else dh
                va = coef * (wl * fb0[i, pl.ds(0, 16)])
                vb = coef * (wh * fb0[i, pl.ds(16, 16)])
                if um2_base is not None:
                    a2, b2 = plsc.unpack(bb0[i, pl.ds(0, 32)], format=ILV)
                    va = va - a2
                    vb = vb - b2
                bb1[i, pl.ds(0, 32)] = plsc.pack(va, vb, format=ILV)
                return 0

            lax.fori_loop(0, nr, rowfn, 0)
            pltpu.sync_copy(bb1.at[sl_nr],
                            uarr.at[pl.ds(out_base + row_off, nr)])

        def compute_u(coef, dpow, s_loader, um2_base, out_base):
            def ch(i, _):
                chunk_body(CCH, r0 + i * CCH, coef, dpow, s_loader,
                           um2_base, out_base)
                return 0

            lax.fori_loop(0, RPT // CCH, ch, 0)
            chunk_body(CTAIL, r0 + (RPT // CCH) * CCH, coef, dpow, s_loader,
                       um2_base, out_base)

        def load_s_from_acc(row_off, nr):
            pltpu.sync_copy(acc.at[pl.ds(row_off, nr)], fb0.at[pl.ds(0, nr)])

        def load_s_from_x(row_off, nr):
            pltpu.sync_copy(x_half.at[pl.ds(row_off, nr)], fb0.at[pl.ds(0, nr)])

        def idx_load(g, sidx, didx):
            @pl.when(g < GPT)
            def _():
                row = c0 + g * R
                pltpu.async_copy(src2_ref.at[pl.ds(row, R)], sidx, isem)
                pltpu.async_copy(dst2_ref.at[pl.ds(row, R)], didx, isem)

        def idx_wait_and_offset(g, sidx, off):
            @pl.when(g < GPT)
            def _():
                pltpu.make_async_copy(dummy_idx, sidx0, isem).wait()
                pltpu.make_async_copy(dummy_idx, sidx0, isem).wait()
                for k in range(R):
                    def aj(j, _, k=k):
                        sl = pl.ds(j * 16, 16)
                        sidx[k, sl] = sidx[k, sl] + off
                        return 0

                    lax.fori_loop(0, PROP_CHUNK // 16, aj, 0)

        def edge_phase(rr):
            off = rr * N
            dummy_b = uarr.at[pl.ds(0, PROP_CHUNK)]

            idx_load(jnp.int32(0), sidx0, didx0)
            idx_wait_and_offset(jnp.int32(0), sidx0, off)
            idx_load(jnp.int32(1), sidx1, didx1)
            for k in range(R):
                pltpu.async_copy(uarr.at[sidx0.at[k]], bbufs[k], gsems[k])

            def convert(k):
                def rowc(j, _):
                    a, b = plsc.unpack(bbufs[k][j, pl.ds(0, 32)], format=ILV)
                    fbufs[k][j, pl.ds(0, 16)] = a
                    fbufs[k][j, pl.ds(16, 16)] = b
                    return 0

                lax.fori_loop(0, PROP_CHUNK, rowc, 0)

            def halfiter(g, sidx_cur, didx_cur, sidx_nxt, didx_nxt):
                for k in range(R):
                    pltpu.make_async_copy(dummy_b, bbufs[k], gsems[k]).wait()

                    @pl.when(g > 0)
                    def _(k=k):
                        pltpu.make_async_copy(dummy_rows, fbufs[k],
                                              ssems[k]).wait()

                    convert(k)
                    pltpu.async_copy(fbufs[k], acc.at[didx_cur.at[k]],
                                     ssems[k], add=True)
                idx_wait_and_offset(g + 1, sidx_nxt, off)

                @pl.when(g + 1 < GPT)
                def _():
                    for k in range(R):
                        pltpu.async_copy(uarr.at[sidx_nxt.at[k]], bbufs[k],
                                         gsems[k])

                idx_load(g + 2, sidx_cur, didx_cur)

            def pair(i, _):
                g = 2 * i
                halfiter(g, sidx0, didx0, sidx1, didx1)
                halfiter(g + 1, sidx1, didx1, sidx0, didx0)
                return 0

            lax.fori_loop(0, GPT // 2, pair, 0)
            for k in range(R):
                pltpu.make_async_copy(dummy_rows, fbufs[k], ssems[k]).wait()

        pltpu.sync_copy(zero_ref.at[myrows], acc.at[myrows])

        @pl.when(s_ax == 0)
        def _():
            pltpu.sync_copy(zero_ref.at[pl.ds(0, ACC_ROWS - N)],
                            acc.at[pl.ds(N, ACC_ROWS - N)])

        compute_u(jnp.float32(1.0), 1, load_s_from_x, None, 0)
        plsc.subcore_barrier()

        def round_fn(rr, _):
            edge_phase(rr)
            plsc.subcore_barrier()
            pltpu.sync_copy(acc.at[myrows], sarr.at[pl.ds(rr * N + r0, RPT)])
            out_base = (rr + 1) * N

            @pl.when(rr == 0)
            def _():
                compute_u(jnp.float32(-1.0), 2, load_s_from_acc,
                          None, out_base)

            @pl.when((rr >= 1) & (rr < 3))
            def _():
                compute_u(jnp.float32(-2.0), 2, load_s_from_acc,
                          (rr - 1) * N, out_base)

            @pl.when(rr < 3)
            def _():
                pltpu.sync_copy(zero_ref.at[myrows], acc.at[myrows])

            plsc.subcore_barrier()
            return 0

        lax.fori_loop(0, 4, round_fn, 0)

    @pl.when(c == 0)
    def _():
        run(xlo_ref, slo_ref, ulo_ref)

    @pl.when(c == 1)
    def _():
        run(xhi_ref, shi_ref, uhi_ref)


_mega_call = pl.kernel(
    _mega_body,
    out_type=[jax.ShapeDtypeStruct((4 * N, H), jnp.float32),
              jax.ShapeDtypeStruct((4 * N, H), jnp.float32),
              jax.ShapeDtypeStruct((4 * N, H), jnp.bfloat16),
              jax.ShapeDtypeStruct((4 * N, H), jnp.bfloat16)],
    mesh=_sc_mesh,
    compiler_params=_sc_params,
    scratch_types=[
        pltpu.VMEM_SHARED((ACC_ROWS, H), jnp.float32),
        pltpu.VMEM((R, PROP_CHUNK), jnp.int32),
        pltpu.VMEM((R, PROP_CHUNK), jnp.int32),
        pltpu.VMEM((R, PROP_CHUNK), jnp.int32),
        pltpu.VMEM((R, PROP_CHUNK), jnp.int32),
        pltpu.VMEM((PROP_CHUNK, H), jnp.bfloat16),
        pltpu.VMEM((PROP_CHUNK, H), jnp.bfloat16),
        pltpu.VMEM((PROP_CHUNK, H), jnp.bfloat16),
        pltpu.VMEM((PROP_CHUNK, H), jnp.bfloat16),
        pltpu.VMEM((PROP_CHUNK, H), jnp.float32),
        pltpu.VMEM((PROP_CHUNK, H), jnp.float32),
        pltpu.VMEM((PROP_CHUNK, H), jnp.float32),
        pltpu.VMEM((PROP_CHUNK, H), jnp.float32),
        pltpu.SemaphoreType.DMA,
        pltpu.SemaphoreType.DMA,
        pltpu.SemaphoreType.DMA,
        pltpu.SemaphoreType.DMA,
        pltpu.SemaphoreType.DMA,
        pltpu.SemaphoreType.DMA,
        pltpu.SemaphoreType.DMA,
        pltpu.SemaphoreType.DMA,
        pltpu.SemaphoreType.DMA,
    ],
)


_row = pl.BlockSpec((BLK, D), lambda i: (i, 0))
_dis_spec = pl.BlockSpec((BLK, 1), lambda i: (i, 0))
_wmat = pl.BlockSpec((D, D), lambda i: (0, 0))
_brow = pl.BlockSpec((1, D), lambda i: (0, 0))


def _shalf_spec(rr):
    return pl.BlockSpec((BLK, H), lambda i, rr=rr: (rr * GRID + i, 0))


def _tc_final_body(x_ref, d32_ref,
                   s1l_ref, s2l_ref, s3l_ref, s4l_ref,
                   s1h_ref, s2h_ref, s3h_ref, s4h_ref,
                   w0_ref, w1_ref, w2_ref, w3_ref, w4_ref, cb_ref,
                   m1_ref, c1_ref, m2_ref, c2_ref, m3_ref, c3_ref,
                   m4_ref, c4_ref, g_ref, bb_ref, y_ref):
    x = x_ref[...]
    d = d32_ref[:, :1]
    s1 = jnp.concatenate([s1l_ref[...], s1h_ref[...]], axis=1)
    s2 = jnp.concatenate([s2l_ref[...], s2h_ref[...]], axis=1)
    s3 = jnp.concatenate([s3l_ref[...], s3h_ref[...]], axis=1)
    s4 = jnp.concatenate([s4l_ref[...], s4h_ref[...]], axis=1)
    tx1 = -(d * s1)
    tx2 = -2.0 * (d * s2) - x
    tx3 = -2.0 * (d * s3) - tx1
    tx4 = -2.0 * (d * s4) - tx2
    out = (jnp.dot(x, w0_ref[...], preferred_element_type=jnp.float32)
           + jnp.dot(tx1, w1_ref[...], preferred_element_type=jnp.float32)
           + jnp.dot(tx2, w2_ref[...], preferred_element_type=jnp.float32)
           + jnp.dot(tx3, w3_ref[...], preferred_element_type=jnp.float32)
           + jnp.dot(tx4, w4_ref[...], preferred_element_type=jnp.float32)
           + cb_ref[...])
    h = jax.nn.relu(jnp.dot(out, m1_ref[...],
                            preferred_element_type=jnp.float32) + c1_ref[...])
    h = jax.nn.relu(jnp.dot(h, m2_ref[...],
                            preferred_element_type=jnp.float32) + c2_ref[...])
    h = jax.nn.relu(jnp.dot(h, m3_ref[...],
                            preferred_element_type=jnp.float32) + c3_ref[...])
    h = jnp.dot(h, m4_ref[...], preferred_element_type=jnp.float32) + c4_ref[...]
    mu = jnp.mean(h, axis=-1, keepdims=True)
    var = jnp.mean((h - mu) ** 2, axis=-1, keepdims=True)
    y_ref[...] = (h - mu) * lax.rsqrt(var + 1e-5) * g_ref[...] + bb_ref[...]


_tc_final = pl.pallas_call(
    _tc_final_body,
    grid=(GRID,),
    in_specs=[_row, pl.BlockSpec((BLK, H), lambda i: (i, 0)),
              _shalf_spec(0), _shalf_spec(1), _shalf_spec(2), _shalf_spec(3),
              _shalf_spec(0), _shalf_spec(1), _shalf_spec(2), _shalf_spec(3),
              _wmat, _wmat, _wmat, _wmat, _wmat, _brow,
              _wmat, _brow, _wmat, _brow, _wmat, _brow, _wmat, _brow,
              _brow, _brow],
    out_specs=_row,
    out_shape=jax.ShapeDtypeStruct((N, D), jnp.float32),
)


def kernel(x_node, edge_index, cheb_W, cheb_b, mlp_W1, mlp_b1, mlp_W2, mlp_b2,
           mlp_W3, mlp_b3, mlp_W4, mlp_b4, ln_g, ln_b):
    ei = edge_index.astype(jnp.int32)
    src = ei[:, 0]
    dst = ei[:, 1]

    dis32 = _deg_call(src)
    zrows = jnp.zeros((N, H), jnp.float32)
    src2 = jnp.concatenate(
        [src, jnp.zeros((EPAD - E,), jnp.int32)]).reshape(EP_CHUNKS, PROP_CHUNK)
    dst2 = jnp.concatenate(
        [dst, jnp.full((EPAD - E,), N, jnp.int32)]).reshape(EP_CHUNKS, PROP_CHUNK)
    xlo = x_node[:, :H]
    xhi = x_node[:, H:]

    sflo, sfhi, _uflo, _ufhi = _mega_call(xlo, xhi, dis32, src2, dst2,
                                          zrows)
    y = _tc_final(x_node, dis32,
                  sflo, sflo, sflo, sflo, sfhi, sfhi, sfhi, sfhi,
                  cheb_W[0], cheb_W[1], cheb_W[2], cheb_W[3], cheb_W[4],
                  cheb_b.reshape(1, D),
                  mlp_W1, mlp_b1.reshape(1, D), mlp_W2, mlp_b2.reshape(1, D),
                  mlp_W3, mlp_b3.reshape(1, D), mlp_W4, mlp_b4.reshape(1, D),
                  ln_g.reshape(1, D), ln_b.reshape(1, D))
    return y


# speedup vs baseline: 1.1834x; 1.1834x over previous
import jax
import jax.numpy as jnp
from jax import lax
from jax.experimental import pallas as pl
from jax.experimental.pallas import tpu as pltpu
from jax.experimental.pallas import tpu_sc as plsc

N = 50000
E = 800000
D = 64
H = D // 2
NSUB = 16
NP = 50176
RPT = N // NSUB
RPT_P = NP // NSUB
EPT = E // NSUB
DEG_CHUNK = 2000
PROP_CHUNK = 128
EP_CHUNKS = 6400
EPAD = EP_CHUNKS * PROP_CHUNK
CPT = EP_CHUNKS // NSUB
R = 5
GPT = CPT // R
ACC_ROWS = N + 8
CCH = 128
CTAIL = RPT - CCH * (RPT // CCH)
BLK = 1000
GRID = N // BLK


def _rsqrt_nr(d):
    i = plsc.bitcast(d, jnp.int32)
    i = jnp.int32(0x5F3759DF) - (i >> 1)
    y = plsc.bitcast(i, jnp.float32)
    for _ in range(4):
        y = y * (1.5 - 0.5 * d * y * y)
    return y


def _deg_body(src_ref, dis_ref, hist, ebuf, tmp, degl, bbuf, hists_sh):
    c = lax.axis_index("c")
    s = lax.axis_index("s")
    zeros16 = jnp.zeros((16,), jnp.float32)
    ones16 = jnp.ones((16,), jnp.float32)

    def zero_hist(i, _):
        hist[pl.ds(i * 16, 16)] = zeros16
        return 0

    lax.fori_loop(0, NP // 16, zero_hist, 0)

    def chunk(ci, _):
        pltpu.sync_copy(src_ref.at[pl.ds(s * EPT + ci * DEG_CHUNK, DEG_CHUNK)], ebuf)

        def inner(j, _):
            idx = ebuf[pl.ds(j * 16, 16)]
            plsc.addupdate_scatter(hist, [idx], ones16)
            return 0

        lax.fori_loop(0, DEG_CHUNK // 16, inner, 0)
        return 0

    lax.fori_loop(0, EPT // DEG_CHUNK, chunk, 0)

    pltpu.sync_copy(hist, hists_sh.at[pl.ds(s * NP, NP)])
    plsc.subcore_barrier()

    r0 = s * RPT_P

    def zero_deg(j, _):
        degl[pl.ds(j * 16, 16)] = zeros16
        return 0

    lax.fori_loop(0, RPT_P // 16, zero_deg, 0)

    def addk(k, _):
        pltpu.sync_copy(hists_sh.at[pl.ds(k * NP + r0, RPT_P)], tmp)

        def aj(j, _):
            sl = pl.ds(j * 16, 16)
            degl[sl] = degl[sl] + tmp[sl]
            return 0

        lax.fori_loop(0, RPT_P // 16, aj, 0)
        return 0

    lax.fori_loop(0, NSUB, addk, 0)

    def dj(j, _):
        sl = pl.ds(j * 16, 16)
        d = degl[sl]
        degl[sl] = jnp.where(d > 0.5, _rsqrt_nr(d), zeros16)
        return 0

    lax.fori_loop(0, RPT_P // 16, dj, 0)

    @pl.when(c == 0)
    def _():
        def wchunk(jc, _):
            def rowf(i, _):
                idxv = jnp.full((16,), jc * 112 + i, jnp.int32)
                dvec = plsc.load_gather(degl, [idxv])
                bbuf[i, pl.ds(0, 16)] = dvec
                bbuf[i, pl.ds(16, 16)] = dvec
                return 0

            lax.fori_loop(0, 112, rowf, 0)
            pltpu.sync_copy(bbuf, dis_ref.at[pl.ds(r0 + jc * 112, 112)])
            return 0

        lax.fori_loop(0, RPT_P // 112, wchunk, 0)


_sc_mesh = plsc.VectorSubcoreMesh(core_axis_name="c", subcore_axis_name="s")
_sc_params = pltpu.CompilerParams(needs_layout_passes=False,
                                  use_tc_tiling_on_sc=False)

_deg_call = pl.kernel(
    _deg_body,
    out_type=jax.ShapeDtypeStruct((NP, H), jnp.float32),
    mesh=_sc_mesh,
    compiler_params=_sc_params,
    scratch_types=[
        pltpu.VMEM((NP,), jnp.float32),
        pltpu.VMEM((DEG_CHUNK,), jnp.int32),
        pltpu.VMEM((RPT_P,), jnp.float32),
        pltpu.VMEM((RPT_P,), jnp.float32),
        pltpu.VMEM((112, H), jnp.float32),
        pltpu.VMEM_SHARED((NSUB * NP,), jnp.float32),
    ],
)


def _mega_body(xlo_ref, xhi_ref, dis32_ref, src2_ref, dst2_ref,
               zero_ref, slo_ref, shi_ref, ulo_ref, uhi_ref,
               acc, sidx0, didx0, sidx1, didx1,
               rb0, rb1, rb2, rb3, rb4,
               gs0, gs1, gs2, gs3, gs4, ss0, ss1, ss2, ss3, ss4, isem):
    c = lax.axis_index("c")
    s_ax = lax.axis_index("s")
    rbufs = (rb0, rb1, rb2, rb3, rb4)
    gsems = (gs0, gs1, gs2, gs3, gs4)
    ssems = (ss0, ss1, ss2, ss3, ss4)
    r0 = s_ax * RPT
    myrows = pl.ds(r0, RPT)
    c0 = s_ax * CPT
    dummy_rows = zero_ref.at[pl.ds(0, PROP_CHUNK)]
    dummy_idx = src2_ref.at[pl.ds(0, R)]

    def run(x_half, sarr, uarr):
        def chunk_body(nr, row_off, coef, dpow, s_loader, um2_base, out_base):
            sl_nr = pl.ds(0, nr)
            s_loader(row_off, nr)
            pltpu.sync_copy(dis32_ref.at[pl.ds(row_off, nr)], rb1.at[sl_nr])
            if um2_base is not None:
                pltpu.sync_copy(uarr.at[pl.ds(um2_base + row_off, nr)],
                                rb2.at[sl_nr])

            def rowfn(i, _):
                for cg in (0, 16):
                    sl = pl.ds(cg, 16)
                    d = rb1[i, sl]
                    w = d * d if dpow == 2 else d
                    v = coef * (w * rb0[i, sl])
                    if um2_base is not None:
                        v = v - rb2[i, sl]
                    rb3[i, sl] = v
                return 0

            lax.fori_loop(0, nr, rowfn, 0)
            pltpu.sync_copy(rb3.at[sl_nr],
                            uarr.at[pl.ds(out_base + row_off, nr)])

        def compute_u(coef, dpow, s_loader, um2_base, out_base):
            def ch(i, _):
                chunk_body(CCH, r0 + i * CCH, coef, dpow, s_loader,
                           um2_base, out_base)
                return 0

            lax.fori_loop(0, RPT // CCH, ch, 0)
            chunk_body(CTAIL, r0 + (RPT // CCH) * CCH, coef, dpow, s_loader,
                       um2_base, out_base)

        def load_s_from_acc(row_off, nr):
            pltpu.sync_copy(acc.at[pl.ds(row_off, nr)], rb0.at[pl.ds(0, nr)])

        def load_s_from_x(row_off, nr):
            pltpu.sync_copy(x_half.at[pl.ds(row_off, nr)], rb0.at[pl.ds(0, nr)])

        def idx_load(g, sidx, didx):
            @pl.when(g < GPT)
            def _():
                row = c0 + g * R
                pltpu.async_copy(src2_ref.at[pl.ds(row, R)], sidx, isem)
                pltpu.async_copy(dst2_ref.at[pl.ds(row, R)], didx, isem)

        def idx_wait_and_offset(g, sidx, off):
            @pl.when(g < GPT)
            def _():
                pltpu.make_async_copy(dummy_idx, sidx0, isem).wait()
                pltpu.make_async_copy(dummy_idx, sidx0, isem).wait()
                for k in range(R):
                    def aj(j, _, k=k):
                        sl = pl.ds(j * 16, 16)
                        sidx[k, sl] = sidx[k, sl] + off
                        return 0

                    lax.fori_loop(0, PROP_CHUNK // 16, aj, 0)

        def edge_phase(rr):
            off = rr * N

            idx_load(jnp.int32(0), sidx0, didx0)
            idx_wait_and_offset(jnp.int32(0), sidx0, off)
            idx_load(jnp.int32(1), sidx1, didx1)
            for k in range(R):
                pltpu.async_copy(uarr.at[sidx0.at[k]], rbufs[k], gsems[k])

            def halfiter(g, sidx_cur, didx_cur, sidx_nxt, didx_nxt):
                for k in range(R):
                    pltpu.make_async_copy(dummy_rows, rbufs[k], gsems[k]).wait()
                    pltpu.async_copy(rbufs[k], acc.at[didx_cur.at[k]],
                                     ssems[k], add=True)
                idx_wait_and_offset(g + 1, sidx_nxt, off)

                @pl.when(g + 1 < GPT)
                def _():
                    for k in range(R):
                        pltpu.make_async_copy(dummy_rows, rbufs[k],
                                              ssems[k]).wait()
                        pltpu.async_copy(uarr.at[sidx_nxt.at[k]], rbufs[k],
                                         gsems[k])

                @pl.when(g + 1 >= GPT)
                def _():
                    for k in range(R):
                        pltpu.make_async_copy(dummy_rows, rbufs[k],
                                              ssems[k]).wait()

                idx_load(g + 2, sidx_cur, didx_cur)

            def pair(i, _):
                g = 2 * i
                halfiter(g, sidx0, didx0, sidx1, didx1)
                halfiter(g + 1, sidx1, didx1, sidx0, didx0)
                return 0

            lax.fori_loop(0, GPT // 2, pair, 0)

        pltpu.sync_copy(zero_ref.at[myrows], acc.at[myrows])

        @pl.when(s_ax == 0)
        def _():
            pltpu.sync_copy(zero_ref.at[pl.ds(0, ACC_ROWS - N)],
                            acc.at[pl.ds(N, ACC_ROWS - N)])

        compute_u(jnp.float32(1.0), 1, load_s_from_x, None, 0)
        plsc.subcore_barrier()

        def round_fn(rr, _):
            edge_phase(rr)
            plsc.subcore_barrier()
            pltpu.sync_copy(acc.at[myrows], sarr.at[pl.ds(rr * N + r0, RPT)])
            out_base = (rr + 1) * N

            @pl.when(rr == 0)
            def _():
                compute_u(jnp.float32(-1.0), 2, load_s_from_acc,
                          None, out_base)

            @pl.when((rr >= 1) & (rr < 3))
            def _():
                compute_u(jnp.float32(-2.0), 2, load_s_from_acc,
                          (rr - 1) * N, out_base)

            @pl.when(rr < 3)
            def _():
                pltpu.sync_copy(zero_ref.at[myrows], acc.at[myrows])

            plsc.subcore_barrier()
            return 0

        lax.fori_loop(0, 4, round_fn, 0)

    @pl.when(c == 0)
    def _():
        run(xlo_ref, slo_ref, ulo_ref)

    @pl.when(c == 1)
    def _():
        run(xhi_ref, shi_ref, uhi_ref)


_mega_call = pl.kernel(
    _mega_body,
    out_type=[jax.ShapeDtypeStruct((4 * N, H), jnp.float32),
              jax.ShapeDtypeStruct((4 * N, H), jnp.float32),
              jax.ShapeDtypeStruct((4 * N, H), jnp.float32),
              jax.ShapeDtypeStruct((4 * N, H), jnp.float32)],
    mesh=_sc_mesh,
    compiler_params=_sc_params,
    scratch_types=[
        pltpu.VMEM_SHARED((ACC_ROWS, H), jnp.float32),
        pltpu.VMEM((R, PROP_CHUNK), jnp.int32),
        pltpu.VMEM((R, PROP_CHUNK), jnp.int32),
        pltpu.VMEM((R, PROP_CHUNK), jnp.int32),
        pltpu.VMEM((R, PROP_CHUNK), jnp.int32),
        pltpu.VMEM((PROP_CHUNK, H), jnp.float32),
        pltpu.VMEM((PROP_CHUNK, H), jnp.float32),
        pltpu.VMEM((PROP_CHUNK, H), jnp.float32),
        pltpu.VMEM((PROP_CHUNK, H), jnp.float32),
        pltpu.VMEM((PROP_CHUNK, H), jnp.float32),
        pltpu.SemaphoreType.DMA,
        pltpu.SemaphoreType.DMA,
        pltpu.SemaphoreType.DMA,
        pltpu.SemaphoreType.DMA,
        pltpu.SemaphoreType.DMA,
        pltpu.SemaphoreType.DMA,
        pltpu.SemaphoreType.DMA,
        pltpu.SemaphoreType.DMA,
        pltpu.SemaphoreType.DMA,
        pltpu.SemaphoreType.DMA,
        pltpu.SemaphoreType.DMA,
    ],
)


_row = pl.BlockSpec((BLK, D), lambda i: (i, 0))
_dis_spec = pl.BlockSpec((BLK, 1), lambda i: (i, 0))
_wmat = pl.BlockSpec((D, D), lambda i: (0, 0))
_brow = pl.BlockSpec((1, D), lambda i: (0, 0))


def _shalf_spec(rr):
    return pl.BlockSpec((BLK, H), lambda i, rr=rr: (rr * GRID + i, 0))


def _tc_final_body(x_ref, d32_ref,
                   s1l_ref, s2l_ref, s3l_ref, s4l_ref,
                   s1h_ref, s2h_ref, s3h_ref, s4h_ref,
                   w0_ref, w1_ref, w2_ref, w3_ref, w4_ref, cb_ref,
                   m1_ref, c1_ref, m2_ref, c2_ref, m3_ref, c3_ref,
                   m4_ref, c4_ref, g_ref, bb_ref, y_ref):
    x = x_ref[...]
    d = d32_ref[:, :1]
    s1 = jnp.concatenate([s1l_ref[...], s1h_ref[...]], axis=1)
    s2 = jnp.concatenate([s2l_ref[...], s2h_ref[...]], axis=1)
    s3 = jnp.concatenate([s3l_ref[...], s3h_ref[...]], axis=1)
    s4 = jnp.concatenate([s4l_ref[...], s4h_ref[...]], axis=1)
    tx1 = -(d * s1)
    tx2 = -2.0 * (d * s2) - x
    tx3 = -2.0 * (d * s3) - tx1
    tx4 = -2.0 * (d * s4) - tx2
    out = (jnp.dot(x, w0_ref[...], preferred_element_type=jnp.float32)
           + jnp.dot(tx1, w1_ref[...], preferred_element_type=jnp.float32)
           + jnp.dot(tx2, w2_ref[...], preferred_element_type=jnp.float32)
           + jnp.dot(tx3, w3_ref[...], preferred_element_type=jnp.float32)
           + jnp.dot(tx4, w4_ref[...], preferred_element_type=jnp.float32)
           + cb_ref[...])
    h = jax.nn.relu(jnp.dot(out, m1_ref[...],
                            preferred_element_type=jnp.float32) + c1_ref[...])
    h = jax.nn.relu(jnp.dot(h, m2_ref[...],
                            preferred_element_type=jnp.float32) + c2_ref[...])
    h = jax.nn.relu(jnp.dot(h, m3_ref[...],
                            preferred_element_type=jnp.float32) + c3_ref[...])
    h = jnp.dot(h, m4_ref[...], preferred_element_type=jnp.float32) + c4_ref[...]
    mu = jnp.mean(h, axis=-1, keepdims=True)
    var = jnp.mean((h - mu) ** 2, axis=-1, keepdims=True)
    y_ref[...] = (h - mu) * lax.rsqrt(var + 1e-5) * g_ref[...] + bb_ref[...]


_tc_final = pl.pallas_call(
    _tc_final_body,
    grid=(GRID,),
    in_specs=[_row, pl.BlockSpec((BLK, H), lambda i: (i, 0)),
              _shalf_spec(0), _shalf_spec(1), _shalf_spec(2), _shalf_spec(3),
              _shalf_spec(0), _shalf_spec(1), _shalf_spec(2), _shalf_spec(3),
              _wmat, _wmat, _wmat, _wmat, _wmat, _brow,
              _wmat, _brow, _wmat, _brow, _wmat, _brow, _wmat, _brow,
              _brow, _brow],
    out_specs=_row,
    out_shape=jax.ShapeDtypeStruct((N, D), jnp.float32),
)


def kernel(x_node, edge_index, cheb_W, cheb_b, mlp_W1, mlp_b1, mlp_W2, mlp_b2,
           mlp_W3, mlp_b3, mlp_W4, mlp_b4, ln_g, ln_b):
    ei = edge_index.astype(jnp.int32)
    src = ei[:, 0]
    dst = ei[:, 1]

    dis32 = _deg_call(src)
    zrows = jnp.zeros((N, H), jnp.float32)
    src2 = jnp.concatenate(
        [src, jnp.zeros((EPAD - E,), jnp.int32)]).reshape(EP_CHUNKS, PROP_CHUNK)
    dst2 = jnp.concatenate(
        [dst, jnp.full((EPAD - E,), N, jnp.int32)]).reshape(EP_CHUNKS, PROP_CHUNK)
    xlo = x_node[:, :H]
    xhi = x_node[:, H:]

    sflo, sfhi, _uflo, _ufhi = _mega_call(xlo, xhi, dis32, src2, dst2,
                                          zrows)
    y = _tc_final(x_node, dis32,
                  sflo, sflo, sflo, sflo, sfhi, sfhi, sfhi, sfhi,
                  cheb_W[0], cheb_W[1], cheb_W[2], cheb_W[3], cheb_W[4],
                  cheb_b.reshape(1, D),
                  mlp_W1, mlp_b1.reshape(1, D), mlp_W2, mlp_b2.reshape(1, D),
                  mlp_W3, mlp_b3.reshape(1, D), mlp_W4, mlp_b4.reshape(1, D),
                  ln_g.reshape(1, D), ln_b.reshape(1, D))
    return y
